# baseline (speedup 1.0000x reference)
import functools

import jax
import jax.numpy as jnp
from jax import lax
from jax.experimental import pallas as pl
from jax.experimental.pallas import tpu as pltpu
from jax.experimental.pallas import tpu_sc as plsc

B = 16384
F = 26
CARD = 100000
D = 16

NC = 2
NS = 16
NW = NC * NS
L = 16
PACK = 8
W = D * PACK

B_PER_W = B // NW
CHUNK = 32
NCHUNK = B_PER_W // CHUNK
GROUPS = CHUNK // L


def _fm_body(x_ref, emb_ref, lin_ref, bias_ref, out_ref,
             xbuf, idx_v, flat_v, rows_v, lin_v, outbuf, bias_v,
             sem_x, sem_emb, sem_lin):
    wid = lax.axis_index("s") * NC + lax.axis_index("c")
    base = wid * B_PER_W

    pltpu.sync_copy(bias_ref, bias_v)
    lane = lax.iota(jnp.int32, L)

    def chunk_body(c, carry):
        cbase = base + c * CHUNK
        pltpu.async_copy(x_ref.at[pl.ds(cbase * F, CHUNK * F)], xbuf,
                         sem_x).wait()

        for f in range(F):
            for j in range(GROUPS):
                bvec = j * L + lane
                raw = plsc.load_gather(xbuf, [bvec * F + f])
                flat = raw + f * CARD
                idx_v[f, pl.ds(j * L, L)] = lax.shift_right_logical(flat, 3)
                flat_v[f, pl.ds(j * L, L)] = flat

        emb_cps = []
        lin_cps = []
        for f in range(F):
            emb_cps.append(pltpu.async_copy(
                emb_ref.at[idx_v.at[f]],
                rows_v.at[pl.ds(f * CHUNK, CHUNK), :], sem_emb))
            lin_cps.append(pltpu.async_copy(
                lin_ref.at[flat_v.at[f]],
                lin_v.at[pl.ds(f * CHUNK, CHUNK)], sem_lin))
        for cp in emb_cps:
            cp.wait()
        for cp in lin_cps:
            cp.wait()

        bias_vec = bias_v[...]

        def group_body(g, gcarry):
            boff = g * L
            bvec = boff + lane
            s = [jnp.zeros((L,), jnp.float32) for _ in range(D)]
            q = [jnp.zeros((L,), jnp.float32) for _ in range(D)]
            lacc = jnp.zeros((L,), jnp.float32)
            for f in range(F):
                ridx = bvec + f * CHUNK
                flat = flat_v[f, pl.ds(boff, L)]
                colb = lax.shift_left(jnp.bitwise_and(flat, 7), 4)
                for d in range(D):
                    v = plsc.load_gather(rows_v, [ridx, colb + d])
                    s[d] = s[d] + v
                    q[d] = q[d] + v * v
                lacc = lacc + plsc.load_gather(lin_v, [ridx])
            inter = jnp.zeros((L,), jnp.float32)
            for d in range(D):
                inter = inter + (s[d] * s[d] - q[d])
            outbuf[pl.ds(boff, L)] = lacc + bias_vec + 0.5 * inter
            return gcarry

        lax.fori_loop(0, GROUPS, group_body, 0)
        pltpu.sync_copy(outbuf, out_ref.at[pl.ds(cbase, CHUNK)])
        return carry

    lax.fori_loop(0, NCHUNK, chunk_body, 0)


@jax.jit
def _fm(x, emb_table, lin2, lin_b):
    mesh = plsc.VectorSubcoreMesh(core_axis_name="c", subcore_axis_name="s")
    return pl.kernel(
        _fm_body,
        out_type=jax.ShapeDtypeStruct((B,), jnp.float32),
        mesh=mesh,
        compiler_params=pltpu.CompilerParams(
            needs_layout_passes=False, use_tc_tiling_on_sc=True),
        scratch_types=[
            pltpu.VMEM((CHUNK * F,), jnp.int32),
            pltpu.VMEM((F, CHUNK), jnp.int32),
            pltpu.VMEM((F, CHUNK), jnp.int32),
            pltpu.VMEM((F * CHUNK, W), jnp.float32),
            pltpu.VMEM((F * CHUNK,), jnp.float32),
            pltpu.VMEM((CHUNK,), jnp.float32),
            pltpu.VMEM((L,), jnp.float32),
            pltpu.SemaphoreType.DMA,
            pltpu.SemaphoreType.DMA,
            pltpu.SemaphoreType.DMA,
        ],
    )(x, emb_table, lin2, lin_b)


def kernel(x, emb_table, lin_w, lin_b):
    bias16 = jnp.broadcast_to(lin_b, (L,))
    out = _fm(x.reshape(B * F), emb_table.reshape(F * CARD // PACK, W),
              lin_w, bias16)
    return out.reshape(B, 1)


# speedup vs baseline: 1.0695x; 1.0695x over previous
import functools

import jax
import jax.numpy as jnp
from jax import lax
from jax.experimental import pallas as pl
from jax.experimental.pallas import tpu as pltpu
from jax.experimental.pallas import tpu_sc as plsc

B = 16384
F = 26
CARD = 100000
D = 16

NC = 2
NS = 16
NW = NC * NS
L = 16

B_PER_W = B // NW
CHUNK = 128
NCHUNK = B_PER_W // CHUNK
GROUPS = CHUNK // L


def _fm_body(x_ref, emb_ref, lin_ref, bias_ref, out_ref,
             idx_v, rows_v, lin_v, outbuf, bias_v,
             sem_x, sem_emb, sem_lin):
    wid = lax.axis_index("s") * NC + lax.axis_index("c")
    base = wid * B_PER_W

    pltpu.sync_copy(bias_ref, bias_v)

    def chunk_body(c, carry):
        cbase = base + c * CHUNK
        x_cps = [pltpu.async_copy(x_ref.at[pl.ds(f * B + cbase, CHUNK)],
                                  idx_v.at[f], sem_x) for f in range(F)]
        for cp in x_cps:
            cp.wait()

        for f in range(1, F):
            for j in range(GROUPS):
                sl = pl.ds(j * L, L)
                idx_v[f, sl] = idx_v[f, sl] + f * CARD

        emb_cps = []
        lin_cps = []
        for f in range(F):
            emb_cps.append(pltpu.async_copy(
                emb_ref.at[idx_v.at[f]],
                rows_v.at[pl.ds(f * CHUNK, CHUNK), :], sem_emb))
            lin_cps.append(pltpu.async_copy(
                lin_ref.at[idx_v.at[f]],
                lin_v.at[pl.ds(f * CHUNK, CHUNK)], sem_lin))
        for cp in emb_cps:
            cp.wait()
        for cp in lin_cps:
            cp.wait()

        bias_vec = bias_v[...]
        lane = lax.iota(jnp.int32, L)

        def group_body(g, gcarry):
            boff = g * L
            bvec = boff + lane
            s = [jnp.zeros((L,), jnp.float32) for _ in range(D)]
            q = [jnp.zeros((L,), jnp.float32) for _ in range(D)]
            lacc = jnp.zeros((L,), jnp.float32)
            for f in range(F):
                ridx = bvec + f * CHUNK
                for d in range(D):
                    dvec = jnp.full((L,), d, jnp.int32)
                    v = plsc.load_gather(rows_v, [ridx, dvec])
                    s[d] = s[d] + v
                    q[d] = q[d] + v * v
                lacc = lacc + plsc.load_gather(lin_v, [ridx])
            inter = jnp.zeros((L,), jnp.float32)
            for d in range(D):
                inter = inter + (s[d] * s[d] - q[d])
            outbuf[pl.ds(boff, L)] = lacc + bias_vec + 0.5 * inter
            return gcarry

        lax.fori_loop(0, GROUPS, group_body, 0)
        pltpu.sync_copy(outbuf, out_ref.at[pl.ds(cbase, CHUNK)])
        return carry

    lax.fori_loop(0, NCHUNK, chunk_body, 0)


@jax.jit
def _fm(x, emb_table, lin2, lin_b):
    mesh = plsc.VectorSubcoreMesh(core_axis_name="c", subcore_axis_name="s")
    return pl.kernel(
        _fm_body,
        out_type=jax.ShapeDtypeStruct((B,), jnp.float32),
        mesh=mesh,
        compiler_params=pltpu.CompilerParams(
            needs_layout_passes=False, use_tc_tiling_on_sc=False),
        scratch_types=[
            pltpu.VMEM((F, CHUNK), jnp.int32),
            pltpu.VMEM((F * CHUNK, D), jnp.float32),
            pltpu.VMEM((F * CHUNK,), jnp.float32),
            pltpu.VMEM((CHUNK,), jnp.float32),
            pltpu.VMEM((L,), jnp.float32),
            pltpu.SemaphoreType.DMA,
            pltpu.SemaphoreType.DMA,
            pltpu.SemaphoreType.DMA,
        ],
    )(x, emb_table, lin2, lin_b)


def kernel(x, emb_table, lin_w, lin_b):
    bias16 = jnp.broadcast_to(lin_b, (L,))
    xf = x.T.reshape(F * B)
    out = _fm(xf, emb_table, lin_w, bias16)
    return out.reshape(B, 1)
